# baseline (speedup 1.0000x reference)
import jax
import jax.numpy as jnp
from jax.experimental import pallas as pl
from jax.experimental.pallas import tpu as pltpu

_BN = 1024
_NOB = 2


def _gating_body(x_ref, w_ref, eq_ref, b_ref, o_hbm, obuf, osem):
    i = pl.program_id(0)
    nsteps = pl.num_programs(0)
    slot = jax.lax.rem(i, _NOB)

    def out_copy(c, s):
        return pltpu.make_async_copy(
            obuf.at[s], o_hbm.at[pl.ds(c * _BN, _BN), :], osem.at[s])

    @pl.when(i >= _NOB)
    def _():
        out_copy(i - _NOB, slot).wait()

    keys = jax.lax.dot_general(
        x_ref[...], w_ref[...],
        dimension_numbers=(((1,), (1,)), ((), ())),
        preferred_element_type=jnp.float32,
    )
    s = (keys + b_ref[0, :][None, :]) * eq_ref[0, :][None, :]
    m = jnp.max(s, axis=-1, keepdims=True)
    e = jnp.exp(s - m)
    obuf[slot] = e / jnp.sum(e, axis=-1, keepdims=True)
    out_copy(i, slot).start()

    @pl.when(i == nsteps - 1)
    def _():
        for k in range(_NOB):
            c = nsteps - _NOB + k
            out_copy(c, jax.lax.rem(c, _NOB)).wait()


def kernel(x, expert_queries, W, b):
    n, d = x.shape
    n_experts = W.shape[0]
    eq2 = expert_queries.reshape(1, n_experts)
    b2 = b.reshape(1, n_experts)
    grid = (n // _BN,)
    return pl.pallas_call(
        _gating_body,
        grid=grid,
        in_specs=[
            pl.BlockSpec((_BN, d), lambda i: (i, 0)),
            pl.BlockSpec((n_experts, d), lambda i: (0, 0)),
            pl.BlockSpec((1, n_experts), lambda i: (0, 0)),
            pl.BlockSpec((1, n_experts), lambda i: (0, 0)),
        ],
        out_specs=pl.BlockSpec(memory_space=pltpu.MemorySpace.HBM),
        out_shape=jax.ShapeDtypeStruct((n, n_experts), jnp.float32),
        scratch_shapes=[
            pltpu.VMEM((_NOB, _BN, n_experts), jnp.float32),
            pltpu.SemaphoreType.DMA((_NOB,)),
        ],
        compiler_params=pltpu.CompilerParams(
            dimension_semantics=("arbitrary",),
        ),
    )(x, W, eq2, b2)
